# baseline (speedup 1.0000x reference)
import functools
import math

import jax
import jax.numpy as jnp
from jax import lax
from jax.experimental import pallas as pl
from jax.experimental.pallas import tpu as pltpu
from jax.experimental.pallas import tpu_sc as plsc

_IDX_PAD = 16


def _sc_gather_body(idx_hbm, table_hbm, out_hbm, idx_v, rows_v, sem):
    cid = lax.axis_index("c")
    sid = lax.axis_index("s")

    @pl.when(jnp.logical_and(cid == 0, sid == 0))
    def _():
        pltpu.sync_copy(idx_hbm, idx_v)
        pltpu.async_copy(table_hbm.at[idx_v], rows_v, sem).wait()
        pltpu.sync_copy(rows_v, out_hbm)


def _sc_gather(idx, table):
    d_model = table.shape[1]
    mesh = plsc.VectorSubcoreMesh(core_axis_name="c", subcore_axis_name="s")
    f = pl.kernel(
        _sc_gather_body,
        mesh=mesh,
        out_type=jax.ShapeDtypeStruct((_IDX_PAD, d_model), jnp.float32),
        scratch_types=[
            pltpu.VMEM((_IDX_PAD,), jnp.int32),
            pltpu.VMEM((_IDX_PAD, d_model), jnp.float32),
            pltpu.SemaphoreType.DMA,
        ],
    )
    return f(idx, table)


def _fast_sin(ang):
    pi_hi = 3.140625
    pi_md = 9.675025939941406e-4
    pi_lo = 1.509957990978376e-7
    n = jnp.round(ang * (1.0 / math.pi))
    r = ang - n * pi_hi
    r = r - n * pi_md
    r = r - n * pi_lo
    r2 = r * r
    p = r2 * -1.9515295891e-4 + 8.3321608736e-3
    p = r2 * p + -1.6666654611e-1
    p = r + r * (r2 * p)
    odd = (n.astype(jnp.int32) & 1) == 1
    return jnp.where(odd, -p, p)


def _make_body(d_model, h, l, dblk, n_dblk, b):
    neg_log = -math.log(10000.0) / d_model
    qrows = dblk // b

    def build_pe_rows(pe4_ref, buf, row0, nrows, d0):
        drow = lax.broadcasted_iota(jnp.int32, (nrows, l), 0) + d0
        lcol = lax.broadcasted_iota(jnp.int32, (nrows, l), 1).astype(jnp.float32)
        dpar = drow & 1
        deven = (drow - dpar).astype(jnp.float32)
        inv_freq = jnp.exp(deven * neg_log)
        ang = lcol * inv_freq + dpar.astype(jnp.float32) * (math.pi / 2)
        pe2 = _fast_sin(ang)
        pe4_ref[buf, pl.ds(row0, nrows)] = jnp.broadcast_to(
            pe2[:, None, :], (nrows, h, l)
        )

    def body(yaw_ref, table_ref, x_ref, o_ref, pe4_ref):
        di = pl.program_id(0)
        bi = pl.program_id(1)

        @pl.when((di == 0) & (bi == 0))
        def _bootstrap():
            build_pe_rows(pe4_ref, 0, 0, dblk, 0)

        @pl.when(di + 1 < n_dblk)
        def _build_next_quarter():
            build_pe_rows(
                pe4_ref, (di + 1) % 2, bi * qrows, qrows,
                (di + 1) * dblk + bi * qrows,
            )

        d0 = di * dblk
        buf = di % 2
        for d_i in range(dblk):
            s = table_ref[bi, d0 + d_i]
            o_ref[0, d_i] = x_ref[0, d_i] + pe4_ref[buf, d_i] + s

    return body


@functools.partial(jax.jit, static_argnums=())
def kernel(x, yaw, yaw_table):
    b, d_model, h, l = x.shape
    dblk = 128
    n_dblk = d_model // dblk

    idx = jnp.zeros((_IDX_PAD,), jnp.int32).at[:b].set(yaw.astype(jnp.int32))
    yemb = _sc_gather(idx, yaw_table)[:b]

    out = pl.pallas_call(
        _make_body(d_model, h, l, dblk, n_dblk, b),
        grid=(n_dblk, b),
        in_specs=[
            pl.BlockSpec(memory_space=pltpu.SMEM),
            pl.BlockSpec(memory_space=pltpu.SMEM),
            pl.BlockSpec((1, dblk, h, l), lambda di, bi: (bi, di, 0, 0)),
        ],
        out_specs=pl.BlockSpec((1, dblk, h, l), lambda di, bi: (bi, di, 0, 0)),
        out_shape=jax.ShapeDtypeStruct((b, d_model, h, l), jnp.float32),
        scratch_shapes=[pltpu.VMEM((2, dblk, h, l), jnp.float32)],
        compiler_params=pltpu.CompilerParams(
            dimension_semantics=("arbitrary", "arbitrary"),
        ),
    )(yaw.astype(jnp.int32), yemb, x)
    return out


# speedup vs baseline: 1.2675x; 1.2675x over previous
import functools
import math

import jax
import jax.numpy as jnp
from jax import lax
from jax.experimental import pallas as pl
from jax.experimental.pallas import tpu as pltpu


def _fast_sin(ang):
    pi_hi = 3.140625
    pi_md = 9.675025939941406e-4
    pi_lo = 1.509957990978376e-7
    n = jnp.round(ang * (1.0 / math.pi))
    r = ang - n * pi_hi
    r = r - n * pi_md
    r = r - n * pi_lo
    r2 = r * r
    p = r2 * -1.9515295891e-4 + 8.3321608736e-3
    p = r2 * p + -1.6666654611e-1
    p = r + r * (r2 * p)
    odd = (n.astype(jnp.int32) & 1) == 1
    return jnp.where(odd, -p, p)


def _make_body(d_model, h, l, dblk, n_dblk, b):
    neg_log = -math.log(10000.0) / d_model
    qrows = dblk // b

    def build_pe_rows(pe4_ref, buf, row0, nrows, d0):
        drow = lax.broadcasted_iota(jnp.int32, (nrows, l), 0) + d0
        lcol = lax.broadcasted_iota(jnp.int32, (nrows, l), 1).astype(jnp.float32)
        dpar = drow & 1
        deven = (drow - dpar).astype(jnp.float32)
        inv_freq = jnp.exp(deven * neg_log)
        ang = lcol * inv_freq + dpar.astype(jnp.float32) * (math.pi / 2)
        pe2 = _fast_sin(ang)
        pe4_ref[buf, pl.ds(row0, nrows)] = jnp.broadcast_to(
            pe2[:, None, :], (nrows, h, l)
        )

    def body(yaw_ref, table_ref, x_ref, o_ref, pe4_ref):
        di = pl.program_id(0)
        bi = pl.program_id(1)

        @pl.when((di == 0) & (bi == 0))
        def _bootstrap():
            build_pe_rows(pe4_ref, 0, 0, dblk, 0)

        @pl.when(di + 1 < n_dblk)
        def _build_next_quarter():
            build_pe_rows(
                pe4_ref, (di + 1) % 2, bi * qrows, qrows,
                (di + 1) * dblk + bi * qrows,
            )

        row = yaw_ref[bi]
        d0 = di * dblk
        buf = di % 2
        for d_i in range(dblk):
            s = table_ref[row, d0 + d_i]
            o_ref[0, d_i] = x_ref[0, d_i] + pe4_ref[buf, d_i] + s

    return body


@functools.partial(jax.jit, static_argnums=())
def kernel(x, yaw, yaw_table):
    b, d_model, h, l = x.shape
    dblk = 128
    n_dblk = d_model // dblk

    out = pl.pallas_call(
        _make_body(d_model, h, l, dblk, n_dblk, b),
        grid=(n_dblk, b),
        in_specs=[
            pl.BlockSpec(memory_space=pltpu.SMEM),
            pl.BlockSpec(memory_space=pltpu.SMEM),
            pl.BlockSpec((1, dblk, h, l), lambda di, bi: (bi, di, 0, 0)),
        ],
        out_specs=pl.BlockSpec((1, dblk, h, l), lambda di, bi: (bi, di, 0, 0)),
        out_shape=jax.ShapeDtypeStruct((b, d_model, h, l), jnp.float32),
        scratch_shapes=[pltpu.VMEM((2, dblk, h, l), jnp.float32)],
        compiler_params=pltpu.CompilerParams(
            dimension_semantics=("arbitrary", "arbitrary"),
        ),
    )(yaw.astype(jnp.int32), yaw_table, x)
    return out
